# baseline (speedup 1.0000x reference)
import functools

import jax
import jax.numpy as jnp
from jax import lax
from jax.experimental import pallas as pl
from jax.experimental.pallas import tpu as pltpu
from jax.experimental.pallas import tpu_sc as plsc

N_ROWS = 32768
D_MODEL = 2048
NC = 2
NS = 16
LANES = 16
NW = NC * NS
ROWS_W = N_ROWS // NW
CHUNK = 8
N_PAIRS = ROWS_W // (2 * CHUNK)
COLV = D_MODEL // LANES


def _sc_body(x_hbm, mask_hbm, g_hbm, b_hbm, out_hbm,
             mask_v, g_v, b_v, tmp32, in0, in1, out0, out1,
             sin0, sin1, sout0, sout1):
    w = lax.axis_index("s") * NC + lax.axis_index("c")
    base = w * ROWS_W

    def _in_copy(buf, sem, r0):
        return pltpu.make_async_copy(
            x_hbm.at[pl.ds(pl.multiple_of(r0, CHUNK), CHUNK)], buf, sem)

    def _out_copy(buf, sem, r0):
        return pltpu.make_async_copy(
            buf, out_hbm.at[pl.ds(pl.multiple_of(r0, CHUNK), CHUNK)], sem)

    _in_copy(in0, sin0, base).start()
    _in_copy(in1, sin1, base + CHUNK).start()
    pltpu.sync_copy(mask_hbm, mask_v)
    pltpu.sync_copy(g_hbm, g_v)
    pltpu.sync_copy(b_hbm, b_v)

    def _red(i, acc):
        for u in range(8):
            off = pl.multiple_of((i * 8 + u) * LANES, LANES)
            acc = jnp.maximum(acc, mask_v[pl.ds(off, LANES)])
        return acc

    accv = lax.fori_loop(0, N_ROWS // (8 * LANES), _red,
                         jnp.zeros((LANES,), jnp.int32))
    tmp32[pl.ds(0, LANES)] = accv
    tmp32[pl.ds(LANES, LANES)] = accv
    for k in range(1, LANES):
        accv = jnp.maximum(accv, tmp32[pl.ds(k, LANES)])
    no_skip = accv[0] == 0

    def _compute(src, dst, mv16, half):
        def _col(c, cc):
            off = pl.multiple_of(c * LANES, LANES)
            g = g_v[pl.ds(off, LANES)]
            b = b_v[pl.ds(off, LANES)]
            for r in range(CHUNK):
                x = src[r, pl.ds(off, LANES)]
                dst[r, pl.ds(off, LANES)] = x * g + b
            return cc

        lax.fori_loop(0, COLV, _col, 0, unroll=2)

        for r in range(CHUNK):
            m = mv16[half * CHUNK + r]

            @pl.when(jnp.logical_or(m != 0, no_skip))
            def _():
                def _cp(c, cc):
                    off = pl.multiple_of(c * LANES, LANES)
                    dst[r, pl.ds(off, LANES)] = src[r, pl.ds(off, LANES)]
                    return cc

                lax.fori_loop(0, COLV, _cp, 0, unroll=4)

    def _pair(pi, carry):
        r0 = base + pi * 2 * CHUNK
        r1 = r0 + CHUNK
        mv16 = mask_v[pl.ds(pl.multiple_of(r0, 2 * CHUNK), 2 * CHUNK)]

        _in_copy(in0, sin0, r0).wait()

        @pl.when(pi > 0)
        def _():
            _out_copy(out0, sout0, r0).wait()

        _compute(in0, out0, mv16, 0)
        _out_copy(out0, sout0, r0).start()

        @pl.when(pi < N_PAIRS - 1)
        def _():
            _in_copy(in0, sin0, r0 + 2 * CHUNK).start()

        _in_copy(in1, sin1, r1).wait()

        @pl.when(pi > 0)
        def _():
            _out_copy(out1, sout1, r1).wait()

        _compute(in1, out1, mv16, 1)
        _out_copy(out1, sout1, r1).start()

        @pl.when(pi < N_PAIRS - 1)
        def _():
            _in_copy(in1, sin1, r1 + 2 * CHUNK).start()

        return carry

    lax.fori_loop(0, N_PAIRS, _pair, 0)
    _out_copy(out0, sout0, base).wait()
    _out_copy(out1, sout1, base).wait()


@functools.partial(
    pl.kernel,
    mesh=plsc.VectorSubcoreMesh(core_axis_name="c", subcore_axis_name="s"),
    out_type=jax.ShapeDtypeStruct((N_ROWS, D_MODEL), jnp.float32),
    scratch_types=[
        pltpu.VMEM((N_ROWS,), jnp.int32),
        pltpu.VMEM((D_MODEL,), jnp.float32),
        pltpu.VMEM((D_MODEL,), jnp.float32),
        pltpu.VMEM((2 * LANES,), jnp.int32),
        pltpu.VMEM((CHUNK, D_MODEL), jnp.float32),
        pltpu.VMEM((CHUNK, D_MODEL), jnp.float32),
        pltpu.VMEM((CHUNK, D_MODEL), jnp.float32),
        pltpu.VMEM((CHUNK, D_MODEL), jnp.float32),
        pltpu.SemaphoreType.DMA,
        pltpu.SemaphoreType.DMA,
        pltpu.SemaphoreType.DMA,
        pltpu.SemaphoreType.DMA,
    ],
)
def _sc_kernel(x_hbm, mask_hbm, g_hbm, b_hbm, out_hbm,
               mask_v, g_v, b_v, tmp32, in0, in1, out0, out1,
               sin0, sin1, sout0, sout1):
    _sc_body(x_hbm, mask_hbm, g_hbm, b_hbm, out_hbm,
             mask_v, g_v, b_v, tmp32, in0, in1, out0, out1,
             sin0, sin1, sout0, sout1)


def kernel(hidden_states, layer_idx, skip_mask, gamma, beta):
    del layer_idx
    mask_i32 = skip_mask.astype(jnp.int32)
    out = _sc_kernel(hidden_states, mask_i32, gamma, beta)
    return (out, skip_mask)


# speedup vs baseline: 1.9463x; 1.9463x over previous
import functools

import jax
import jax.numpy as jnp
from jax import lax
from jax.experimental import pallas as pl
from jax.experimental.pallas import tpu as pltpu
from jax.experimental.pallas import tpu_sc as plsc

N_ROWS = 32768
D_MODEL = 2048
NC = 2
NS = 16
LANES = 16
NW = NC * NS
ROWS_W = N_ROWS // NW
CHUNK = 8
N_PAIRS = ROWS_W // (2 * CHUNK)
COLV = D_MODEL // LANES


def _sc_body(x_hbm, mask_hbm, g_hbm, b_hbm, out_hbm,
             mask_v, g_v, b_v, tmp32, in0, in1, out0, out1,
             sin0, sin1, sout0, sout1):
    w = lax.axis_index("s") * NC + lax.axis_index("c")
    base = w * ROWS_W

    def _in_copy(buf, sem, r0):
        return pltpu.make_async_copy(
            x_hbm.at[pl.ds(pl.multiple_of(r0, CHUNK), CHUNK)], buf, sem)

    def _out_copy(buf, sem, r0):
        return pltpu.make_async_copy(
            buf, out_hbm.at[pl.ds(pl.multiple_of(r0, CHUNK), CHUNK)], sem)

    _in_copy(in0, sin0, base).start()
    _in_copy(in1, sin1, base + CHUNK).start()
    pltpu.sync_copy(mask_hbm, mask_v)
    pltpu.sync_copy(g_hbm, g_v)
    pltpu.sync_copy(b_hbm, b_v)

    @plsc.parallel_loop(0, N_ROWS // LANES, carry=jnp.zeros((LANES,), jnp.int32))
    def accv(i, acc):
        return jnp.maximum(acc, mask_v[pl.ds(i * LANES, LANES)])
    tmp32[pl.ds(0, LANES)] = accv
    tmp32[pl.ds(LANES, LANES)] = accv
    for k in range(1, LANES):
        accv = jnp.maximum(accv, tmp32[pl.ds(k, LANES)])
    no_skip = accv[0] == 0

    def _compute(src, dst, mv16, half):
        @plsc.parallel_loop(0, COLV, unroll=2)
        def _col(c):
            off = pl.multiple_of(c * LANES, LANES)
            g = g_v[pl.ds(off, LANES)]
            b = b_v[pl.ds(off, LANES)]
            for r in range(CHUNK):
                x = src[r, pl.ds(off, LANES)]
                dst[r, pl.ds(off, LANES)] = x * g + b

        for r in range(CHUNK):
            m = mv16[half * CHUNK + r]

            @pl.when(jnp.logical_or(m != 0, no_skip))
            def _():
                @plsc.parallel_loop(0, COLV, unroll=4)
                def _cp(c):
                    off = pl.multiple_of(c * LANES, LANES)
                    dst[r, pl.ds(off, LANES)] = src[r, pl.ds(off, LANES)]

    def _pair(pi, carry):
        r0 = base + pi * 2 * CHUNK
        r1 = r0 + CHUNK
        mv16 = mask_v[pl.ds(pl.multiple_of(r0, 2 * CHUNK), 2 * CHUNK)]

        _in_copy(in0, sin0, r0).wait()

        @pl.when(pi > 0)
        def _():
            _out_copy(out0, sout0, r0).wait()

        _compute(in0, out0, mv16, 0)
        _out_copy(out0, sout0, r0).start()

        @pl.when(pi < N_PAIRS - 1)
        def _():
            _in_copy(in0, sin0, r0 + 2 * CHUNK).start()

        _in_copy(in1, sin1, r1).wait()

        @pl.when(pi > 0)
        def _():
            _out_copy(out1, sout1, r1).wait()

        _compute(in1, out1, mv16, 1)
        _out_copy(out1, sout1, r1).start()

        @pl.when(pi < N_PAIRS - 1)
        def _():
            _in_copy(in1, sin1, r1 + 2 * CHUNK).start()

        return carry

    lax.fori_loop(0, N_PAIRS, _pair, 0)
    _out_copy(out0, sout0, base).wait()
    _out_copy(out1, sout1, base).wait()


@functools.partial(
    pl.kernel,
    mesh=plsc.VectorSubcoreMesh(core_axis_name="c", subcore_axis_name="s"),
    out_type=jax.ShapeDtypeStruct((N_ROWS, D_MODEL), jnp.float32),
    scratch_types=[
        pltpu.VMEM((N_ROWS,), jnp.int32),
        pltpu.VMEM((D_MODEL,), jnp.float32),
        pltpu.VMEM((D_MODEL,), jnp.float32),
        pltpu.VMEM((2 * LANES,), jnp.int32),
        pltpu.VMEM((CHUNK, D_MODEL), jnp.float32),
        pltpu.VMEM((CHUNK, D_MODEL), jnp.float32),
        pltpu.VMEM((CHUNK, D_MODEL), jnp.float32),
        pltpu.VMEM((CHUNK, D_MODEL), jnp.float32),
        pltpu.SemaphoreType.DMA,
        pltpu.SemaphoreType.DMA,
        pltpu.SemaphoreType.DMA,
        pltpu.SemaphoreType.DMA,
    ],
)
def _sc_kernel(x_hbm, mask_hbm, g_hbm, b_hbm, out_hbm,
               mask_v, g_v, b_v, tmp32, in0, in1, out0, out1,
               sin0, sin1, sout0, sout1):
    _sc_body(x_hbm, mask_hbm, g_hbm, b_hbm, out_hbm,
             mask_v, g_v, b_v, tmp32, in0, in1, out0, out1,
             sin0, sin1, sout0, sout1)


def kernel(hidden_states, layer_idx, skip_mask, gamma, beta):
    del layer_idx
    mask_i32 = skip_mask.astype(jnp.int32)
    out = _sc_kernel(hidden_states, mask_i32, gamma, beta)
    return (out, skip_mask)
